# baseline (speedup 1.0000x reference)
import functools

import jax
import jax.numpy as jnp
from jax import lax
from jax.experimental import pallas as pl
from jax.experimental.pallas import tpu as pltpu
from jax.experimental.pallas import tpu_sc as plsc

_NBUF = 4


def _transpose_rows(src, dst_v, h, base16, n_rows):
    rvecs = [base16 + (16 * k) for k in range(4)]
    unroll = 8
    assert n_rows % unroll == 0

    def row_body(mi, carry):
        m0 = mi * unroll
        for dm in range(unroll):
            m = m0 + dm
            c0 = jnp.full((16,), 2 * m, jnp.int32)
            c1 = c0 + 1
            for k in range(4):
                dst_v[h, m, pl.ds(16 * k, 16)] = plsc.load_gather(src, [rvecs[k], c0])
                dst_v[h, m, pl.ds(64 + 16 * k, 16)] = plsc.load_gather(src, [rvecs[k], c1])
        return carry

    lax.fori_loop(0, n_rows // unroll, row_body, 0)


def _make_transpose(d_model, vocab, n_cores, n_subcores):
    n_workers = n_cores * n_subcores
    n_full = vocab // 128
    rem = vocab - n_full * 128
    t_full = n_full // n_workers
    extras = n_full - n_workers * t_full
    assert t_full % 2 == 0 and t_full >= 4
    assert rem % 2 == 0

    mesh = plsc.VectorSubcoreMesh(core_axis_name="c", subcore_axis_name="s")

    @functools.partial(
        pl.kernel,
        mesh=mesh,
        out_type=jax.ShapeDtypeStruct((vocab * d_model // 128, 128), jnp.float32),
        scratch_types=[
            pltpu.VMEM((2, d_model, 129), jnp.float32),
            pltpu.VMEM((2, d_model, 128), jnp.float32),
            pltpu.SemaphoreType.DMA((2,)),
            pltpu.SemaphoreType.DMA((2,)),
        ],
        compiler_params=pltpu.CompilerParams(use_tc_tiling_on_sc=True, needs_layout_passes=False),
    )
    def k(wt_hbm, out_hbm, src_v, dst_v, src_sem, dst_sem):
        wid = lax.axis_index("s") * n_cores + lax.axis_index("c")
        base16 = jnp.arange(16, dtype=jnp.int32)

        def blk(t):
            return t * n_workers + wid

        def src_copy(t, h):
            return pltpu.make_async_copy(
                wt_hbm.at[:, pl.ds(blk(t) * 128, 128)],
                src_v.at[h, :, pl.ds(0, 128)],
                src_sem.at[h],
            )

        def dst_copy(t, h):
            return pltpu.make_async_copy(
                dst_v.at[h],
                out_hbm.at[pl.ds(blk(t) * 64, 64)],
                dst_sem.at[h],
            )

        def process_t(t, h, first=False, last=False):
            src_copy(t, h).wait()
            if not first:
                dst_copy(t - 2, h).wait()
            _transpose_rows(src_v.at[h], dst_v, h, base16, 64)
            if not last:
                src_copy(t + 2, h).start()
            dst_copy(t, h).start()

        src_copy(0, 0).start()
        src_copy(1, 1).start()
        process_t(0, 0, first=True)
        process_t(1, 1, first=True)

        def body(u, carry):
            process_t(2 * u, 0)
            process_t(2 * u + 1, 1)
            return carry

        lax.fori_loop(1, t_full // 2 - 1, body, 0)

        process_t(t_full - 2, 0, last=True)
        process_t(t_full - 1, 1, last=True)
        dst_copy(t_full - 2, 0).wait()
        dst_copy(t_full - 1, 1).wait()

        if extras:
            @pl.when(wid < extras)
            def _():
                j2 = n_workers * t_full + wid
                pltpu.sync_copy(wt_hbm.at[:, pl.ds(j2 * 128, 128)], src_v.at[0, :, pl.ds(0, 128)])
                _transpose_rows(src_v.at[0], dst_v, 0, base16, 64)
                pltpu.sync_copy(dst_v.at[0], out_hbm.at[pl.ds(j2 * 64, 64)])

        if rem:
            @pl.when(wid == extras)
            def _():
                for d in range(d_model):
                    pltpu.make_async_copy(
                        wt_hbm.at[d, pl.ds(n_full * 128, rem)],
                        src_v.at[1, d, pl.ds(0, rem)],
                        src_sem.at[1],
                    ).start()
                for d in range(d_model):
                    pltpu.make_async_copy(
                        wt_hbm.at[d, pl.ds(n_full * 128, rem)],
                        src_v.at[1, d, pl.ds(0, rem)],
                        src_sem.at[1],
                    ).wait()
                _transpose_rows(src_v.at[1], dst_v, 1, base16, rem // 2)
                pltpu.sync_copy(
                    dst_v.at[1, pl.ds(0, rem // 2)],
                    out_hbm.at[pl.ds(n_full * 64, rem // 2)],
                )

    return k


def _make_gather(n_tokens, d_model, n_cores, n_subcores):
    n_workers = n_cores * n_subcores
    tok_per_w = n_tokens // n_workers
    n_chunks = tok_per_w // 64
    n_groups = n_chunks // _NBUF
    assert n_groups % 2 == 0 and n_groups >= 4

    mesh = plsc.VectorSubcoreMesh(core_axis_name="c", subcore_axis_name="s")

    @functools.partial(
        pl.kernel,
        mesh=mesh,
        out_type=jax.ShapeDtypeStruct((2 * n_tokens, d_model), jnp.float32),
        scratch_types=[
            pltpu.VMEM((tok_per_w,), jnp.int32),
            pltpu.VMEM((2, _NBUF, 128), jnp.int32),
            pltpu.VMEM((2, _NBUF, 128, d_model), jnp.float32),
            pltpu.SemaphoreType.DMA((2, _NBUF)),
            pltpu.SemaphoreType.DMA((2, _NBUF)),
        ],
        compiler_params=pltpu.CompilerParams(use_tc_tiling_on_sc=False, needs_layout_passes=False),
    )
    def k(ids_hbm, table_hbm, out_hbm, idx_v, dup_v, rows_v, sem_g, sem_s):
        wid = lax.axis_index("s") * n_cores + lax.axis_index("c")
        tok0 = wid * tok_per_w
        base16 = jnp.arange(16, dtype=jnp.int32)
        pltpu.sync_copy(ids_hbm.at[pl.ds(tok0, tok_per_w)], idx_v)

        def chunk_of(g, b):
            return g * _NBUF + b

        def build_dup(g, h, b):
            c = chunk_of(g, b)
            for k16 in range(8):
                pos = ((base16 + 16 * k16) >> 1) + 64 * c
                dup_v[h, b, pl.ds(16 * k16, 16)] = plsc.load_gather(idx_v, [pos])

        def gather_copy(g, h, b):
            return pltpu.make_async_copy(
                table_hbm.at[dup_v.at[h, b]],
                rows_v.at[h, b],
                sem_g.at[h, b],
            )

        def store_copy(g, h, b):
            return pltpu.make_async_copy(
                rows_v.at[h, b],
                out_hbm.at[pl.ds(2 * tok0 + chunk_of(g, b) * 128, 128)],
                sem_s.at[h, b],
            )

        def process(g, h, first=False, last=False):
            nh = 1 - h
            if not first:
                for b in range(_NBUF):
                    store_copy(g - 1, nh, b).wait()
            if not last:
                for b in range(_NBUF):
                    build_dup(g + 1, nh, b)
                    gather_copy(g + 1, nh, b).start()
            for b in range(_NBUF):
                gather_copy(g, h, b).wait()
                store_copy(g, h, b).start()

        for b in range(_NBUF):
            build_dup(0, 0, b)
            gather_copy(0, 0, b).start()
        process(0, 0, first=True)
        process(1, 1)

        def body(u, carry):
            g2 = 2 * u
            process(g2, 0)
            process(g2 + 1, 1)
            return carry

        lax.fori_loop(1, n_groups // 2 - 1, body, 0)

        process(n_groups - 2, 0)
        process(n_groups - 1, 1, last=True)
        for b in range(_NBUF):
            store_copy(n_groups - 1, 1, b).wait()

    return k


def kernel(token_ids, weights):
    batch, seq_len = token_ids.shape
    vocab, d_model = weights.shape
    n = batch * seq_len
    ids = token_ids.astype(jnp.int32).reshape(n)

    info = plsc.get_sparse_core_info()
    nc, ns = info.num_cores, info.num_subcores
    k1 = _make_transpose(d_model, vocab, nc, ns)
    k2 = _make_gather(n, d_model, nc, ns)

    scratch = k1(weights.T)
    table = scratch.reshape(vocab, d_model)
    out2 = k2(ids, table)
    out = out2.reshape(n, 2 * d_model)[:, :d_model]
    return out.reshape(batch, seq_len, d_model)


# speedup vs baseline: 1.0916x; 1.0916x over previous
import functools

import jax
import jax.numpy as jnp
from jax import lax
from jax.experimental import pallas as pl
from jax.experimental.pallas import tpu as pltpu
from jax.experimental.pallas import tpu_sc as plsc

_NBUF = 4


def _make_tc_transpose(d_model, vocab, blk):
    n_blocks = (vocab + blk - 1) // blk
    sub = blk // 128
    out_rows = n_blocks * (blk // 2)

    def body(in_ref, out_ref):
        for s in range(sub):
            xa = in_ref[:, 128 * s : 128 * s + 64]
            xb = in_ref[:, 128 * s + 64 : 128 * s + 128]
            out_ref[64 * s : 64 * (s + 1), 0:d_model] = xa.T
            out_ref[64 * s : 64 * (s + 1), d_model : 2 * d_model] = xb.T

    return pl.pallas_call(
        body,
        grid=(n_blocks,),
        in_specs=[pl.BlockSpec((d_model, blk), lambda j: (0, j))],
        out_specs=pl.BlockSpec((blk // 2, 2 * d_model), lambda j: (j, 0)),
        out_shape=jax.ShapeDtypeStruct((out_rows, 2 * d_model), jnp.float32),
    )


def _make_gather(n_tokens, d_model, n_cores, n_subcores):
    n_workers = n_cores * n_subcores
    tok_per_w = n_tokens // n_workers
    n_chunks = tok_per_w // 64
    n_groups = n_chunks // _NBUF
    assert n_groups % 2 == 0 and n_groups >= 4

    mesh = plsc.VectorSubcoreMesh(core_axis_name="c", subcore_axis_name="s")

    @functools.partial(
        pl.kernel,
        mesh=mesh,
        out_type=jax.ShapeDtypeStruct((2 * n_tokens, d_model), jnp.float32),
        scratch_types=[
            pltpu.VMEM((tok_per_w,), jnp.int32),
            pltpu.VMEM((2, _NBUF, 128), jnp.int32),
            pltpu.VMEM((2, _NBUF, 128, d_model), jnp.float32),
            pltpu.SemaphoreType.DMA((2, _NBUF)),
            pltpu.SemaphoreType.DMA((2, _NBUF)),
        ],
        compiler_params=pltpu.CompilerParams(use_tc_tiling_on_sc=False, needs_layout_passes=False),
    )
    def k(ids_hbm, table_hbm, out_hbm, idx_v, dup_v, rows_v, sem_g, sem_s):
        wid = lax.axis_index("s") * n_cores + lax.axis_index("c")
        tok0 = wid * tok_per_w
        base16 = jnp.arange(16, dtype=jnp.int32)
        pltpu.sync_copy(ids_hbm.at[pl.ds(tok0, tok_per_w)], idx_v)

        def chunk_of(g, b):
            return g * _NBUF + b

        def build_dup(g, h, b):
            c = chunk_of(g, b)
            for k16 in range(8):
                pos = ((base16 + 16 * k16) >> 1) + 64 * c
                v = plsc.load_gather(idx_v, [pos])
                row = (((v >> 7) << 6) + (v & 63)) * 2 + ((v >> 6) & 1)
                dup_v[h, b, pl.ds(16 * k16, 16)] = row

        def gather_copy(g, h, b):
            return pltpu.make_async_copy(
                table_hbm.at[dup_v.at[h, b]],
                rows_v.at[h, b],
                sem_g.at[h, b],
            )

        def store_copy(g, h, b):
            return pltpu.make_async_copy(
                rows_v.at[h, b],
                out_hbm.at[pl.ds(2 * tok0 + chunk_of(g, b) * 128, 128)],
                sem_s.at[h, b],
            )

        def process(g, h, first=False, last=False):
            nh = 1 - h
            if not first:
                for b in range(_NBUF):
                    store_copy(g - 1, nh, b).wait()
            if not last:
                for b in range(_NBUF):
                    build_dup(g + 1, nh, b)
                    gather_copy(g + 1, nh, b).start()
            for b in range(_NBUF):
                gather_copy(g, h, b).wait()
                store_copy(g, h, b).start()

        for b in range(_NBUF):
            build_dup(0, 0, b)
            gather_copy(0, 0, b).start()
        process(0, 0, first=True)
        process(1, 1)

        def body(u, carry):
            g2 = 2 * u
            process(g2, 0)
            process(g2 + 1, 1)
            return carry

        lax.fori_loop(1, n_groups // 2 - 1, body, 0)

        process(n_groups - 2, 0)
        process(n_groups - 1, 1, last=True)
        for b in range(_NBUF):
            store_copy(n_groups - 1, 1, b).wait()

    return k


def kernel(token_ids, weights):
    batch, seq_len = token_ids.shape
    vocab, d_model = weights.shape
    n = batch * seq_len
    ids = token_ids.astype(jnp.int32).reshape(n)

    info = plsc.get_sparse_core_info()
    nc, ns = info.num_cores, info.num_subcores
    k1 = _make_tc_transpose(d_model, vocab, 512)
    k2 = _make_gather(n, d_model, nc, ns)

    scratch = k1(weights.T)
    table = scratch.reshape(scratch.shape[0] * 2, d_model)
    out2 = k2(ids, table)
    out = out2.reshape(n, 2 * d_model)[:, :d_model]
    return out.reshape(batch, seq_len, d_model)


# speedup vs baseline: 1.0942x; 1.0024x over previous
import functools

import jax
import jax.numpy as jnp
from jax import lax
from jax.experimental import pallas as pl
from jax.experimental.pallas import tpu as pltpu
from jax.experimental.pallas import tpu_sc as plsc

_NBUF = 4


def _make_tc_transpose(d_model, vocab, blk):
    n_blocks = (vocab + blk - 1) // blk
    sub = blk // 128
    out_rows = n_blocks * (blk // 2)

    def body(in_ref, out_ref):
        ident = (
            lax.broadcasted_iota(jnp.int32, (d_model, d_model), 0)
            == lax.broadcasted_iota(jnp.int32, (d_model, d_model), 1)
        ).astype(jnp.float32)

        def tr(x):
            return lax.dot_general(
                x, ident, (((0,), (0,)), ((), ())),
                preferred_element_type=jnp.float32,
            )

        for s in range(sub):
            xa = in_ref[:, 128 * s : 128 * s + 64]
            xb = in_ref[:, 128 * s + 64 : 128 * s + 128]
            out_ref[64 * s : 64 * (s + 1), 0:d_model] = tr(xa)
            out_ref[64 * s : 64 * (s + 1), d_model : 2 * d_model] = tr(xb)

    return pl.pallas_call(
        body,
        grid=(n_blocks,),
        in_specs=[pl.BlockSpec((d_model, blk), lambda j: (0, j))],
        out_specs=pl.BlockSpec((blk // 2, 2 * d_model), lambda j: (j, 0)),
        out_shape=jax.ShapeDtypeStruct((out_rows, 2 * d_model), jnp.float32),
    )


def _make_gather(n_tokens, d_model, n_cores, n_subcores):
    n_workers = n_cores * n_subcores
    tok_per_w = n_tokens // n_workers
    n_chunks = tok_per_w // 64
    n_groups = n_chunks // _NBUF
    assert n_groups % 2 == 0 and n_groups >= 4

    mesh = plsc.VectorSubcoreMesh(core_axis_name="c", subcore_axis_name="s")

    @functools.partial(
        pl.kernel,
        mesh=mesh,
        out_type=jax.ShapeDtypeStruct((2 * n_tokens, d_model), jnp.float32),
        scratch_types=[
            pltpu.VMEM((tok_per_w,), jnp.int32),
            pltpu.VMEM((2, _NBUF, 128), jnp.int32),
            pltpu.VMEM((2, _NBUF, 128, d_model), jnp.float32),
            pltpu.SemaphoreType.DMA((2, _NBUF)),
            pltpu.SemaphoreType.DMA((2, _NBUF)),
        ],
        compiler_params=pltpu.CompilerParams(use_tc_tiling_on_sc=False, needs_layout_passes=False),
    )
    def k(ids_hbm, table_hbm, out_hbm, idx_v, dup_v, rows_v, sem_g, sem_s):
        wid = lax.axis_index("s") * n_cores + lax.axis_index("c")
        tok0 = wid * tok_per_w
        base16 = jnp.arange(16, dtype=jnp.int32)
        pltpu.sync_copy(ids_hbm.at[pl.ds(tok0, tok_per_w)], idx_v)

        def chunk_of(g, b):
            return g * _NBUF + b

        def build_dup(g, h, b):
            c = chunk_of(g, b)
            for k16 in range(8):
                pos = ((base16 + 16 * k16) >> 1) + 64 * c
                v = plsc.load_gather(idx_v, [pos])
                row = (((v >> 7) << 6) + (v & 63)) * 2 + ((v >> 6) & 1)
                dup_v[h, b, pl.ds(16 * k16, 16)] = row

        def gather_copy(g, h, b):
            return pltpu.make_async_copy(
                table_hbm.at[dup_v.at[h, b]],
                rows_v.at[h, b],
                sem_g.at[h, b],
            )

        def store_copy(g, h, b):
            return pltpu.make_async_copy(
                rows_v.at[h, b],
                out_hbm.at[pl.ds(2 * tok0 + chunk_of(g, b) * 128, 128)],
                sem_s.at[h, b],
            )

        def process(g, h, first=False, last=False):
            nh = 1 - h
            if not first:
                for b in range(_NBUF):
                    store_copy(g - 1, nh, b).wait()
            if not last:
                for b in range(_NBUF):
                    build_dup(g + 1, nh, b)
                    gather_copy(g + 1, nh, b).start()
            for b in range(_NBUF):
                gather_copy(g, h, b).wait()
                store_copy(g, h, b).start()

        for b in range(_NBUF):
            build_dup(0, 0, b)
            gather_copy(0, 0, b).start()
        process(0, 0, first=True)
        process(1, 1)

        def body(u, carry):
            g2 = 2 * u
            process(g2, 0)
            process(g2 + 1, 1)
            return carry

        lax.fori_loop(1, n_groups // 2 - 1, body, 0)

        process(n_groups - 2, 0)
        process(n_groups - 1, 1, last=True)
        for b in range(_NBUF):
            store_copy(n_groups - 1, 1, b).wait()

    return k


def kernel(token_ids, weights):
    batch, seq_len = token_ids.shape
    vocab, d_model = weights.shape
    n = batch * seq_len
    ids = token_ids.astype(jnp.int32).reshape(n)

    info = plsc.get_sparse_core_info()
    nc, ns = info.num_cores, info.num_subcores
    k1 = _make_tc_transpose(d_model, vocab, 512)
    k2 = _make_gather(n, d_model, nc, ns)

    scratch = k1(weights.T)
    table = scratch.reshape(scratch.shape[0] * 2, d_model)
    out2 = k2(ids, table)
    out = out2.reshape(n, 2 * d_model)[:, :d_model]
    return out.reshape(batch, seq_len, d_model)


# speedup vs baseline: 1.9728x; 1.8030x over previous
import functools

import jax
import jax.numpy as jnp
from jax import lax
from jax.experimental import pallas as pl
from jax.experimental.pallas import tpu as pltpu
from jax.experimental.pallas import tpu_sc as plsc

_LANES = 128
_NBUF = 2


def _make_sc_gather(n_rows, lanes, width, n_cores, n_subcores):
    n_workers = n_cores * n_subcores
    rows_per_w = n_rows // n_workers
    n_groups = rows_per_w // _NBUF
    assert n_groups % 2 == 0 and n_groups >= 4

    mesh = plsc.VectorSubcoreMesh(core_axis_name="c", subcore_axis_name="s")

    @functools.partial(
        pl.kernel,
        mesh=mesh,
        out_type=jax.ShapeDtypeStruct((n_rows * lanes, width), jnp.float32),
        scratch_types=[
            pltpu.VMEM((rows_per_w, lanes), jnp.int32),
            pltpu.VMEM((2, _NBUF, lanes, width), jnp.float32),
            pltpu.SemaphoreType.DMA((2, _NBUF)),
            pltpu.SemaphoreType.DMA((2, _NBUF)),
        ],
        compiler_params=pltpu.CompilerParams(use_tc_tiling_on_sc=False),
    )
    def k(ids_hbm, table_hbm, out_hbm, idx_v, rows_v, sem_g, sem_s):
        wid = lax.axis_index("s") * n_cores + lax.axis_index("c")
        row0 = wid * rows_per_w
        pltpu.sync_copy(ids_hbm.at[pl.ds(row0, rows_per_w)], idx_v)

        def gather_copy(g, h, b):
            return pltpu.make_async_copy(
                table_hbm.at[idx_v.at[g * _NBUF + b]],
                rows_v.at[h, b],
                sem_g.at[h, b],
            )

        def store_copy(g, h, b):
            return pltpu.make_async_copy(
                rows_v.at[h, b],
                out_hbm.at[pl.ds((row0 + g * _NBUF + b) * lanes, lanes)],
                sem_s.at[h, b],
            )

        def process(g, h, first=False, last=False):
            nh = 1 - h
            if not first:
                for b in range(_NBUF):
                    store_copy(g - 1, nh, b).wait()
            if not last:
                for b in range(_NBUF):
                    gather_copy(g + 1, nh, b).start()
            for b in range(_NBUF):
                gather_copy(g, h, b).wait()
                store_copy(g, h, b).start()

        for b in range(_NBUF):
            gather_copy(0, 0, b).start()
        process(0, 0, first=True)
        process(1, 1)

        def body(t, carry):
            g2 = 2 * t
            process(g2, 0)
            process(g2 + 1, 1)
            return carry

        lax.fori_loop(1, n_groups // 2 - 1, body, 0)

        process(n_groups - 2, 0)
        process(n_groups - 1, 1, last=True)
        for b in range(_NBUF):
            store_copy(n_groups - 1, 1, b).wait()

    return k


def kernel(token_ids, weights):
    batch, seq_len = token_ids.shape
    vocab, d_model = weights.shape
    n = batch * seq_len
    n_rows = n // _LANES
    ids2d = token_ids.astype(jnp.int32).reshape(n_rows, _LANES)
    wp = jnp.pad(weights, ((0, 0), (0, _LANES - d_model)))

    info = plsc.get_sparse_core_info()
    k = _make_sc_gather(n_rows, _LANES, _LANES, info.num_cores, info.num_subcores)
    out = k(ids2d, wp)
    return out[:, :d_model].reshape(batch, seq_len, d_model)
